# baseline (speedup 1.0000x reference)
import functools

import jax
import jax.numpy as jnp
from jax import lax
from jax.experimental import pallas as pl
from jax.experimental.pallas import tpu as pltpu
from jax.experimental.pallas import tpu_sc as plsc

_B, _T, _S, _D = 8, 16, 576, 768
_NC, _NS = 2, 16
_NW = _NC * _NS
_CR = 32
_CPF = _S // _CR
_NCH = _B * _T * _CPF
_CPW = _NCH // _NW
_MAXQ = 16
_TB = 4


def _sc_fill_body(p32_hbm, tok_hbm, out_hbm, tokbuf, pvec, sem):
    w = lax.axis_index("s") * _NC + lax.axis_index("c")

    pltpu.sync_copy(p32_hbm, pvec)
    pltpu.sync_copy(tok_hbm, tokbuf)

    n_out = jnp.int32(0)
    for i in range(_CPW):
        g = w + _NW * i
        f = g // _CPF
        c = g - f * _CPF
        b = f // _T
        t = f - b * _T
        pb = pvec[pl.ds(b, 16)][0]
        fill = (t // _TB) * _TB + (_TB - 1) < pb

        @pl.when(jnp.logical_and(fill, n_out >= _MAXQ))
        def _():
            pltpu.make_async_copy(tok_hbm, tokbuf, sem).wait()

        n_out = jnp.where(fill, jnp.minimum(n_out, _MAXQ - 1), n_out)

        @pl.when(fill)
        def _():
            pltpu.async_copy(tokbuf, out_hbm.at[b, t, pl.ds(c * _CR, _CR)], sem)

        n_out = n_out + jnp.where(fill, 1, 0).astype(jnp.int32)

    def _drain(_, carry):
        pltpu.make_async_copy(tok_hbm, tokbuf, sem).wait()
        return carry

    lax.fori_loop(0, n_out, _drain, jnp.int32(0))


@functools.partial(jax.jit, static_argnums=())
def _sc_fill(p32, tokchunk):
    fn = pl.kernel(
        _sc_fill_body,
        out_type=jax.ShapeDtypeStruct((_B, _T, _S, _D), jnp.float32),
        mesh=plsc.VectorSubcoreMesh(core_axis_name="c", subcore_axis_name="s"),
        scratch_types=[
            pltpu.VMEM((_CR, _D), jnp.float32),
            pltpu.VMEM((32,), jnp.int32),
            pltpu.SemaphoreType.DMA,
        ],
    )
    return fn(p32, tokchunk)


def _tc_body(pref, x_ref, tok_ref, o0_ref, o_ref):
    b = pl.program_id(0)
    j = pl.program_id(1)
    p = pref[b]
    for tt in range(_TB):
        masked = j * _TB + tt < p

        @pl.when(masked)
        def _():
            o_ref[0, tt] = jnp.broadcast_to(tok_ref[...], (_S, _D))

        @pl.when(jnp.logical_not(masked))
        def _():
            o_ref[0, tt] = x_ref[0, tt]


def _fold_index_map(b, j, pref):
    p = pref[b]
    fully_masked = j * _TB + _TB - 1 < p
    j_sel = jnp.where(fully_masked, jnp.minimum(p // _TB, _T // _TB - 1), j)
    return b, j_sel, 0, 0


def _tc_pass(prefix_len, x, tok2d, out0):
    grid_spec = pltpu.PrefetchScalarGridSpec(
        num_scalar_prefetch=1,
        grid=(_B, _T // _TB),
        in_specs=[
            pl.BlockSpec((1, _TB, _S, _D), _fold_index_map),
            pl.BlockSpec((1, _D), lambda b, j, pref: (0, 0)),
            pl.BlockSpec(memory_space=pl.ANY),
        ],
        out_specs=pl.BlockSpec((1, _TB, _S, _D), _fold_index_map),
    )
    fn = pl.pallas_call(
        _tc_body,
        grid_spec=grid_spec,
        out_shape=jax.ShapeDtypeStruct(x.shape, x.dtype),
        input_output_aliases={3: 0},
    )
    return fn(prefix_len, x, tok2d, out0)


def kernel(x, prefix_len, dropout_mask_token):
    p32 = jnp.zeros((32,), jnp.int32).at[:_B].set(prefix_len)
    tokchunk = jnp.broadcast_to(dropout_mask_token[None, :], (_CR, _D))
    out0 = _sc_fill(p32, tokchunk)
    return _tc_pass(prefix_len, x, dropout_mask_token.reshape(1, _D), out0)


# speedup vs baseline: 1.0857x; 1.0857x over previous
import functools

import jax
import jax.numpy as jnp
from jax import lax
from jax.experimental import pallas as pl
from jax.experimental.pallas import tpu as pltpu
from jax.experimental.pallas import tpu_sc as plsc

_B, _T, _S, _D = 8, 16, 576, 768
_NC, _NS = 2, 16
_NW = _NC * _NS
_CR = 32
_CPF = _S // _CR
_NCH = _B * _T * _CPF
_CPW = _NCH // _NW
_NSL = 4
_AHEAD = 2


def _sc_body(x_hbm, p32_hbm, tok_hbm, out_hbm, tokbuf, stag, pvec, sem_g, sem_s):
    w = lax.axis_index("s") * _NC + lax.axis_index("c")

    pltpu.sync_copy(p32_hbm, pvec)
    pltpu.sync_copy(tok_hbm, tokbuf)

    def params(i):
        g = w + _NW * i
        f = g // _CPF
        c = g - f * _CPF
        b = f // _T
        t = f - b * _T
        pb = pvec[pl.ds(b, 16)][0]
        return b, t, c, t < pb

    def gather(i, prm):
        b, t, c, masked = prm

        @pl.when(jnp.logical_not(masked))
        def _():
            pltpu.async_copy(
                x_hbm.at[b, t, pl.ds(c * _CR, _CR)], stag.at[i % _NSL], sem_g
            )

    prm = [params(i) for i in range(_AHEAD)]
    for i in range(_AHEAD):
        gather(i, prm[i])

    for i in range(_CPW):
        b, t, c, masked = prm[i % _AHEAD]
        if i >= 2:
            pltpu.make_async_copy(x_hbm.at[0, 0, pl.ds(0, _CR)], stag.at[0], sem_s).wait()
        if i + _AHEAD < _CPW:
            nxt = params(i + _AHEAD)
            gather(i + _AHEAD, nxt)
            prm[i % _AHEAD] = nxt
        dst = out_hbm.at[b, t, pl.ds(c * _CR, _CR)]

        @pl.when(masked)
        def _():
            pltpu.async_copy(tokbuf, dst, sem_s)

        @pl.when(jnp.logical_not(masked))
        def _():
            pltpu.make_async_copy(x_hbm.at[0, 0, pl.ds(0, _CR)], stag.at[0], sem_g).wait()
            pltpu.async_copy(stag.at[i % _NSL], dst, sem_s)

    for i in range(2):
        pltpu.make_async_copy(x_hbm.at[0, 0, pl.ds(0, _CR)], stag.at[0], sem_s).wait()


@functools.partial(jax.jit, static_argnums=())
def _sc_call(x, p32, tokchunk):
    fn = pl.kernel(
        _sc_body,
        out_type=jax.ShapeDtypeStruct((_B, _T, _S, _D), jnp.float32),
        mesh=plsc.VectorSubcoreMesh(core_axis_name="c", subcore_axis_name="s"),
        scratch_types=[
            pltpu.VMEM((_CR, _D), jnp.float32),
            pltpu.VMEM((_NSL, _CR, _D), jnp.float32),
            pltpu.VMEM((32,), jnp.int32),
            pltpu.SemaphoreType.DMA,
            pltpu.SemaphoreType.DMA,
        ],
    )
    return fn(x, p32, tokchunk)


def kernel(x, prefix_len, dropout_mask_token):
    p32 = jnp.zeros((32,), jnp.int32).at[:_B].set(prefix_len)
    tokchunk = jnp.broadcast_to(dropout_mask_token[None, :], (_CR, _D))
    return _sc_call(x, p32, tokchunk)
